# Initial kernel scaffold; baseline (speedup 1.0000x reference)
import jax
import jax.numpy as jnp
from jax.experimental import pallas as pl


def kernel(proposal_boxes, gt_boxes, gt_classes):
    raise NotImplementedError("write your pallas kernel here")


# speedup vs baseline: 2.0314x; 2.0314x over previous
import functools

import jax
import jax.numpy as jnp
import numpy as np
from jax.experimental import pallas as pl

NUM_CLASSES = 80
BATCH = 512
NUM_FG = 128
NUM_BG = 384
IOU_T = 0.5
C_BLK = 512
GT_PAD = 512
N_WORKERS = 16


@functools.lru_cache(maxsize=None)
def _perm_const(total, npad, gt0_pos):
    with jax.ensure_compile_time_eval():
        rnd = np.asarray(jax.random.uniform(jax.random.key(42), (total,)))
    perm = np.argsort(-rnd, kind="stable").astype(np.int32)
    pad = np.full((npad - total,), gt0_pos, np.int32)
    return np.concatenate([perm, pad])


def _match_body(bt_ref, gt_ref, vals_ref, idxs_ref):
    px1 = bt_ref[0:1, :]
    py1 = bt_ref[1:2, :]
    px2 = bt_ref[2:3, :]
    py2 = bt_ref[3:4, :]
    gx1 = gt_ref[:, 0:1]
    gy1 = gt_ref[:, 1:2]
    gx2 = gt_ref[:, 2:3]
    gy2 = gt_ref[:, 3:4]
    area_g = gt_ref[:, 4:5]
    area_p = (px2 - px1) * (py2 - py1)
    w = jnp.clip(jnp.minimum(gx2, px2) - jnp.maximum(gx1, px1), 0.0)
    h = jnp.clip(jnp.minimum(gy2, py2) - jnp.maximum(gy1, py1), 0.0)
    inter = w * h
    union = area_g + area_p - inter
    iou = jnp.where(union > 0, inter / union, 0.0)
    mx = jnp.max(iou, axis=0, keepdims=True)
    row = jax.lax.broadcasted_iota(jnp.int32, iou.shape, 0)
    idx = jnp.min(
        jnp.where(iou == mx, row, jnp.int32(GT_PAD)), axis=0, keepdims=True
    )
    vals_ref[...] = mx
    idxs_ref[...] = idx


def _match(btp, gtc, npad):
    grid = npad // C_BLK
    return pl.pallas_call(
        _match_body,
        grid=(grid,),
        in_specs=[
            pl.BlockSpec((8, C_BLK), lambda i: (0, i)),
            pl.BlockSpec((GT_PAD, 8), lambda i: (0, 0)),
        ],
        out_specs=[
            pl.BlockSpec((1, C_BLK), lambda i: (0, i)),
            pl.BlockSpec((1, C_BLK), lambda i: (0, i)),
        ],
        out_shape=[
            jax.ShapeDtypeStruct((1, npad), jnp.float32),
            jax.ShapeDtypeStruct((1, npad), jnp.int32),
        ],
    )(btp, gtc)


def kernel(proposal_boxes, gt_boxes, gt_classes):
    n = proposal_boxes.shape[0]
    m = gt_boxes.shape[0]
    total = n + m
    npad = ((total + N_WORKERS * C_BLK - 1) // (N_WORKERS * C_BLK)) * (
        N_WORKERS * C_BLK
    )

    perm = jnp.asarray(_perm_const(total, npad, n))

    boxes_all = jnp.concatenate([proposal_boxes, gt_boxes], axis=0)
    boxes_p = jnp.take(boxes_all, perm, axis=0)
    btp = jnp.concatenate(
        [boxes_p.T, jnp.zeros((4, npad), jnp.float32)], axis=0
    )

    gt_pad = jnp.zeros((GT_PAD, 8), jnp.float32)
    gt_pad = gt_pad.at[:m, 0:4].set(gt_boxes)
    area_g = (gt_boxes[:, 2] - gt_boxes[:, 0]) * (gt_boxes[:, 3] - gt_boxes[:, 1])
    gtc = gt_pad.at[:m, 4].set(area_g)

    vals_p, idxs_p = _match(btp, gtc, npad)
    vals_p = vals_p[0]
    idxs_p = idxs_p[0]

    pos = vals_p >= IOU_T
    j = jnp.arange(npad, dtype=jnp.int32)
    _, fg_j = jax.lax.top_k(jnp.where(pos, -j, -npad - 1), NUM_FG)
    _, bg_j = jax.lax.top_k(jnp.where(~pos, -j, -npad - 1), NUM_BG)
    sel_j = jnp.concatenate([fg_j, bg_j])
    sampled_idxs = jnp.take(perm, sel_j)
    sampled_vals = jnp.take(vals_p, sel_j)
    gidx = jnp.take(idxs_p, sel_j)
    gidx = jnp.minimum(gidx, m - 1)
    sampled_boxes = jnp.take(boxes_p, sel_j, axis=0)
    sampled_gt = jnp.take(gt_boxes, gidx, axis=0)
    gcls = jnp.take(gt_classes, gidx)
    sampled_cls = jnp.where(sampled_vals >= IOU_T, gcls, NUM_CLASSES)
    out = jnp.concatenate(
        [sampled_boxes, sampled_gt, sampled_vals[:, None]], axis=1
    )
    return out, sampled_idxs, sampled_cls


# speedup vs baseline: 2.5901x; 1.2750x over previous
import functools

import jax
import jax.numpy as jnp
import numpy as np
from jax import lax
from jax.experimental import pallas as pl
from jax.experimental.pallas import tpu as pltpu
from jax.experimental.pallas import tpu_sc as plsc

NUM_CLASSES = 80
BATCH = 512
NUM_FG = 128
NUM_BG = 384
IOU_T = 0.5
C_BLK = 512
GT_PAD = 512
NSUB = 16


def _threefry_uniform(seed, n):
    k1 = np.uint32(seed >> 32)
    k2 = np.uint32(seed & 0xFFFFFFFF)
    ks = [k1, k2, np.uint32(k1 ^ k2 ^ np.uint32(0x1BD11BDA))]
    x0 = np.zeros(n, np.uint32)
    x1 = np.arange(n, dtype=np.uint32)
    rotations = [[13, 15, 26, 6], [17, 29, 16, 24]]

    def rotl(v, d):
        return (v << np.uint32(d)) | (v >> np.uint32(32 - d))

    with np.errstate(over="ignore"):
        x0 = x0 + ks[0]
        x1 = x1 + ks[1]
        for i in range(5):
            for r in rotations[i % 2]:
                x0 = x0 + x1
                x1 = rotl(x1, r)
                x1 = x1 ^ x0
            x0 = x0 + ks[(i + 1) % 3]
            x1 = x1 + ks[(i + 2) % 3] + np.uint32(i + 1)
    bits = x0 ^ x1
    fl = ((bits >> np.uint32(9)) | np.uint32(0x3F800000)).view(np.float32)
    return np.maximum(np.float32(0.0), fl - np.float32(1.0))


@functools.lru_cache(maxsize=None)
def _perm_const(total, npad, gt0_pos):
    rnd = _threefry_uniform(42, total)
    perm = np.argsort(-rnd, kind="stable").astype(np.int32)
    pad = np.full((npad - total,), gt0_pos, np.int32)
    return np.concatenate([perm, pad])


def _match_body(bt_ref, gt_ref, vals_ref, idxs_ref):
    px1 = bt_ref[0:1, :]
    py1 = bt_ref[1:2, :]
    px2 = bt_ref[2:3, :]
    py2 = bt_ref[3:4, :]
    gx1 = gt_ref[:, 0:1]
    gy1 = gt_ref[:, 1:2]
    gx2 = gt_ref[:, 2:3]
    gy2 = gt_ref[:, 3:4]
    area_g = gt_ref[:, 4:5]
    area_p = (px2 - px1) * (py2 - py1)
    w = jnp.clip(jnp.minimum(gx2, px2) - jnp.maximum(gx1, px1), 0.0)
    h = jnp.clip(jnp.minimum(gy2, py2) - jnp.maximum(gy1, py1), 0.0)
    inter = w * h
    union = area_g + area_p - inter
    iou = jnp.where(union > 0, inter / union, 0.0)
    mx = jnp.max(iou, axis=0, keepdims=True)
    row = jax.lax.broadcasted_iota(jnp.int32, iou.shape, 0)
    idx = jnp.min(
        jnp.where(iou == mx, row, jnp.int32(GT_PAD)), axis=0, keepdims=True
    )
    vals_ref[...] = mx
    idxs_ref[...] = idx


def _match(btp, gtc, npad):
    grid = npad // C_BLK
    return pl.pallas_call(
        _match_body,
        grid=(grid,),
        in_specs=[
            pl.BlockSpec((8, C_BLK), lambda i: (0, i)),
            pl.BlockSpec((GT_PAD, 8), lambda i: (0, 0)),
        ],
        out_specs=[
            pl.BlockSpec((1, C_BLK), lambda i: (0, i)),
            pl.BlockSpec((1, C_BLK), lambda i: (0, i)),
        ],
        out_shape=[
            jax.ShapeDtypeStruct((1, npad), jnp.float32),
            jax.ShapeDtypeStruct((1, npad), jnp.int32),
        ],
    )(btp, gtc)


def _make_sampler(npad, m):
    ch = npad // NSUB
    nv = ch // 16
    per_w = BATCH // NSUB
    mesh = plsc.VectorSubcoreMesh(core_axis_name="c", subcore_axis_name="s")

    @functools.partial(
        pl.kernel,
        mesh=mesh,
        out_type=[
            jax.ShapeDtypeStruct((9 * BATCH,), jnp.float32),
            jax.ShapeDtypeStruct((BATCH,), jnp.int32),
            jax.ShapeDtypeStruct((BATCH,), jnp.int32),
        ],
        scratch_types=[
            pltpu.VMEM((ch,), jnp.float32),
            pltpu.VMEM((16,), jnp.int32),
            pltpu.VMEM((16,), jnp.float32),
            pltpu.VMEM((16,), jnp.int32),
            pltpu.VMEM((NSUB, 16), jnp.int32),
            pltpu.VMEM((BATCH + 16,), jnp.int32),
            pltpu.VMEM((2 * 16,), jnp.int32),
            pltpu.VMEM_SHARED((NSUB, 16), jnp.int32),
            pltpu.VMEM_SHARED((NSUB * (BATCH + 16),), jnp.int32),
            pltpu.SemaphoreType.DMA,
        ],
        compiler_params=pltpu.CompilerParams(needs_layout_passes=False),
    )
    def sampler(
        vals_hbm,
        idxs_hbm,
        perm_hbm,
        bflat_hbm,
        gtflat_hbm,
        gtcls_hbm,
        out_t,
        sidx,
        scls,
        vchunk,
        ibuf,
        fbuf,
        sbuf,
        cbuf,
        sel_local,
        rbuf,
        shared,
        sel_rows,
        sem,
    ):
        wid = lax.axis_index("s")
        lane = lax.iota(jnp.int32, 16)

        def gath_f(tbl, idxv):
            pltpu.async_copy(tbl.at[idxv], fbuf, sem).wait()
            return fbuf[...]

        def gath_i(tbl, idxv):
            pltpu.async_copy(tbl.at[idxv], ibuf, sem).wait()
            return ibuf[...]

        pltpu.sync_copy(vals_hbm.at[pl.ds(wid * ch, ch)], vchunk)

        def count_body(i, cp):
            v = vchunk[pl.ds(i * 16, 16)]
            return cp + jnp.sum(jnp.where(v >= IOU_T, 1, 0))

        cp = lax.fori_loop(0, nv, count_body, jnp.int32(0))
        cnt = jnp.where(lane == 0, cp, jnp.where(lane == 1, ch - cp, 0))
        ibuf[...] = cnt
        pltpu.sync_copy(ibuf, shared.at[wid])
        plsc.subcore_barrier()

        pltpu.sync_copy(shared, cbuf)
        acc = jnp.zeros((16,), jnp.int32)
        for r in range(NSUB):
            acc = acc + jnp.where(r < wid, cbuf[r, :], 0)
        fg_base = acc[0]
        bg_base = acc[1]

        zero = jnp.zeros((16,), jnp.int32)
        for k in range((BATCH + 16) // 16):
            sel_local[pl.ds(k * 16, 16)] = zero

        @pl.when(jnp.logical_or(fg_base < NUM_FG, bg_base < NUM_BG))
        def _():
            def scan_body(i, carry):
                cps, cns = carry
                v = vchunk[pl.ds(i * 16, 16)]
                mask = v >= IOU_T
                mi = jnp.where(mask, 1, 0)
                csp = jnp.cumsum(mi)
                csn = jnp.cumsum(1 - mi)
                posd = cps + csp - 1
                negd = cns + csn - 1 + NUM_FG
                sel_pos = jnp.logical_and(mask, posd < NUM_FG)
                sel_neg = jnp.logical_and(~mask, negd < BATCH)
                selm = jnp.logical_or(sel_pos, sel_neg)
                dest = jnp.where(
                    sel_pos, posd, jnp.where(sel_neg, negd, BATCH + lane)
                )
                plsc.store_scatter(
                    sel_local, [dest], wid * ch + i * 16 + lane, mask=selm
                )
                npos = csp[15]
                return (cps + npos, cns + 16 - npos)

            lax.fori_loop(0, nv, scan_body, (fg_base, bg_base))

        pltpu.sync_copy(sel_local, sel_rows.at[pl.ds(wid * (BATCH + 16), BATCH + 16)])
        plsc.subcore_barrier()

        acc0 = zero
        acc1 = zero
        for r in range(NSUB):
            pltpu.sync_copy(
                sel_rows.at[pl.ds(r * (BATCH + 16) + wid * per_w, per_w)], rbuf
            )
            acc0 = acc0 + rbuf[pl.ds(0, 16)]
            acc1 = acc1 + rbuf[pl.ds(16, 16)]
        for b in range(per_w // 16):
            off = wid * per_w + b * 16
            jv = acc0 if b == 0 else acc1
            jv = jnp.clip(jv, 0, npad - 1)
            val = gath_f(vals_hbm, jv)
            gidx = gath_i(idxs_hbm, jv)
            gidx = jnp.clip(gidx, 0, GT_PAD - 1)
            oidx = gath_i(perm_hbm, jv)
            ibuf[...] = oidx
            pltpu.sync_copy(ibuf, sidx.at[pl.ds(off, 16)])
            gcls = gath_i(gtcls_hbm, gidx)
            cls = jnp.where(val >= IOU_T, gcls, NUM_CLASSES)
            ibuf[...] = cls
            pltpu.sync_copy(ibuf, scls.at[pl.ds(off, 16)])
            for r in range(4):
                bx = gath_f(bflat_hbm, jv + r * npad)
                fbuf[...] = bx
                pltpu.sync_copy(fbuf, out_t.at[pl.ds(r * BATCH + off, 16)])
            for r in range(4):
                gx = gath_f(gtflat_hbm, gidx + r * GT_PAD)
                fbuf[...] = gx
                pltpu.sync_copy(fbuf, out_t.at[pl.ds((4 + r) * BATCH + off, 16)])
            fbuf[...] = val
            pltpu.sync_copy(fbuf, out_t.at[pl.ds(8 * BATCH + off, 16)])

    return sampler


def kernel(proposal_boxes, gt_boxes, gt_classes):
    n = proposal_boxes.shape[0]
    m = gt_boxes.shape[0]
    total = n + m
    blk = NSUB * C_BLK
    npad = ((total + blk - 1) // blk) * blk

    perm = jnp.asarray(_perm_const(total, npad, n))

    boxes_all = jnp.concatenate([proposal_boxes, gt_boxes], axis=0)
    boxes_p = jnp.take(boxes_all, perm, axis=0)
    btp = jnp.concatenate(
        [boxes_p.T, jnp.zeros((4, npad), jnp.float32)], axis=0
    )

    gt_pad = jnp.zeros((GT_PAD, 8), jnp.float32)
    gt_pad = gt_pad.at[:m, 0:4].set(gt_boxes)
    area_g = (gt_boxes[:, 2] - gt_boxes[:, 0]) * (gt_boxes[:, 3] - gt_boxes[:, 1])
    gtc = gt_pad.at[:m, 4].set(area_g)

    vals_p, idxs_p = _match(btp, gtc, npad)
    vals_p = vals_p.reshape(npad)
    idxs_p = idxs_p.reshape(npad)

    bflat = btp[:4].reshape(4 * npad)
    gtflat = gt_pad[:, 0:4].T.reshape(4 * GT_PAD)
    gtcls = jnp.zeros((GT_PAD,), jnp.int32).at[:m].set(gt_classes)

    sampler = _make_sampler(npad, m)
    out_t, sampled_idxs, sampled_cls = sampler(
        vals_p, idxs_p, perm, bflat, gtflat, gtcls
    )
    return out_t.reshape(9, BATCH).T, sampled_idxs, sampled_cls


# speedup vs baseline: 5.1120x; 1.9737x over previous
import functools

import jax
import jax.numpy as jnp
import numpy as np
from jax import lax
from jax.experimental import pallas as pl
from jax.experimental.pallas import tpu as pltpu
from jax.experimental.pallas import tpu_sc as plsc

NUM_CLASSES = 80
BATCH = 512
NUM_FG = 128
NUM_BG = 384
IOU_T = 0.5
C_BLK = 512
GT_PAD = 512
NSUB = 16


def _threefry_uniform(seed, n):
    k1 = np.uint32(seed >> 32)
    k2 = np.uint32(seed & 0xFFFFFFFF)
    ks = [k1, k2, np.uint32(k1 ^ k2 ^ np.uint32(0x1BD11BDA))]
    x0 = np.zeros(n, np.uint32)
    x1 = np.arange(n, dtype=np.uint32)
    rotations = [[13, 15, 26, 6], [17, 29, 16, 24]]

    def rotl(v, d):
        return (v << np.uint32(d)) | (v >> np.uint32(32 - d))

    with np.errstate(over="ignore"):
        x0 = x0 + ks[0]
        x1 = x1 + ks[1]
        for i in range(5):
            for r in rotations[i % 2]:
                x0 = x0 + x1
                x1 = rotl(x1, r)
                x1 = x1 ^ x0
            x0 = x0 + ks[(i + 1) % 3]
            x1 = x1 + ks[(i + 2) % 3] + np.uint32(i + 1)
    bits = x0 ^ x1
    fl = ((bits >> np.uint32(9)) | np.uint32(0x3F800000)).view(np.float32)
    return np.maximum(np.float32(0.0), fl - np.float32(1.0))


@functools.lru_cache(maxsize=None)
def _perm_const(total, npad, gt0_pos):
    rnd = _threefry_uniform(42, total)
    perm = np.argsort(-rnd, kind="stable").astype(np.int32)
    pad = np.full((npad - total,), gt0_pos, np.int32)
    return np.concatenate([perm, pad])


def _match_body(bt_ref, gt_ref, vals_ref, idxs_ref):
    px1 = bt_ref[0:1, :]
    py1 = bt_ref[1:2, :]
    px2 = bt_ref[2:3, :]
    py2 = bt_ref[3:4, :]
    gx1 = gt_ref[:, 0:1]
    gy1 = gt_ref[:, 1:2]
    gx2 = gt_ref[:, 2:3]
    gy2 = gt_ref[:, 3:4]
    area_g = gt_ref[:, 4:5]
    area_p = (px2 - px1) * (py2 - py1)
    w = jnp.clip(jnp.minimum(gx2, px2) - jnp.maximum(gx1, px1), 0.0)
    h = jnp.clip(jnp.minimum(gy2, py2) - jnp.maximum(gy1, py1), 0.0)
    inter = w * h
    union = area_g + area_p - inter
    iou = jnp.where(union > 0, inter / union, 0.0)
    mx = jnp.max(iou, axis=0, keepdims=True)
    row = jax.lax.broadcasted_iota(jnp.int32, iou.shape, 0)
    idx = jnp.min(
        jnp.where(iou == mx, row, jnp.int32(GT_PAD)), axis=0, keepdims=True
    )
    vals_ref[...] = mx
    idxs_ref[...] = idx


def _match(btp, gtc, npad):
    grid = npad // C_BLK
    return pl.pallas_call(
        _match_body,
        grid=(grid,),
        in_specs=[
            pl.BlockSpec((8, C_BLK), lambda i: (0, i)),
            pl.BlockSpec((GT_PAD, 8), lambda i: (0, 0)),
        ],
        out_specs=[
            pl.BlockSpec((1, C_BLK), lambda i: (0, i)),
            pl.BlockSpec((1, C_BLK), lambda i: (0, i)),
        ],
        out_shape=[
            jax.ShapeDtypeStruct((1, npad), jnp.float32),
            jax.ShapeDtypeStruct((1, npad), jnp.int32),
        ],
    )(btp, gtc)


def _make_sampler(npad, m):
    ch = npad // NSUB
    nv = ch // 16
    ng = ch // 128
    per_w = BATCH // NSUB
    mesh = plsc.VectorSubcoreMesh(core_axis_name="c", subcore_axis_name="s")

    @functools.partial(
        pl.kernel,
        mesh=mesh,
        out_type=[
            jax.ShapeDtypeStruct((9 * BATCH,), jnp.float32),
            jax.ShapeDtypeStruct((BATCH,), jnp.int32),
            jax.ShapeDtypeStruct((BATCH,), jnp.int32),
        ],
        scratch_types=[
            pltpu.VMEM((ch,), jnp.int32),
            pltpu.VMEM((ch,), jnp.float32),
            pltpu.VMEM((16,), jnp.int32),
            pltpu.VMEM((16,), jnp.float32),
            pltpu.VMEM((NSUB, 16), jnp.int32),
            pltpu.VMEM((BATCH + 16,), jnp.int32),
            pltpu.VMEM((2 * 16,), jnp.int32),
            pltpu.VMEM_SHARED((NSUB, 16), jnp.int32),
            pltpu.VMEM_SHARED((NSUB * (BATCH + 16),), jnp.int32),
            pltpu.SemaphoreType.DMA,
        ],
        compiler_params=pltpu.CompilerParams(needs_layout_passes=False),
    )
    def sampler(
        vals_hbm,
        idxs_hbm,
        perm_hbm,
        bflat_hbm,
        gtflat_hbm,
        gtcls_hbm,
        out_t,
        sidx,
        scls,
        pchunk,
        vchunk,
        ibuf,
        fbuf,
        cbuf,
        sel_local,
        rbuf,
        shared,
        sel_rows,
        sem,
    ):
        wid = lax.axis_index("s")
        lane = lax.iota(jnp.int32, 16)

        def gath_f(tbl, idxv):
            pltpu.async_copy(tbl.at[idxv], fbuf, sem).wait()
            return fbuf[...]

        def gath_i(tbl, idxv):
            pltpu.async_copy(tbl.at[idxv], ibuf, sem).wait()
            return ibuf[...]

        pltpu.sync_copy(perm_hbm.at[pl.ds(wid * ch, ch)], pchunk)

        def gather_body(t, _):
            pltpu.async_copy(
                vals_hbm.at[pchunk.at[pl.ds(t * 128, 128)]],
                vchunk.at[pl.ds(t * 128, 128)],
                sem,
            ).wait()
            return 0

        lax.fori_loop(0, ng, gather_body, 0)

        def count_body(i, cp):
            v = vchunk[pl.ds(i * 16, 16)]
            return cp + jnp.sum(jnp.where(v >= IOU_T, 1, 0))

        cp = lax.fori_loop(0, nv, count_body, jnp.int32(0))
        cnt = jnp.where(lane == 0, cp, jnp.where(lane == 1, ch - cp, 0))
        ibuf[...] = cnt
        pltpu.sync_copy(ibuf, shared.at[wid])
        plsc.subcore_barrier()

        pltpu.sync_copy(shared, cbuf)
        acc = jnp.zeros((16,), jnp.int32)
        for r in range(NSUB):
            acc = acc + jnp.where(r < wid, cbuf[r, :], 0)
        fg_base = acc[0]
        bg_base = acc[1]

        zero = jnp.zeros((16,), jnp.int32)
        for k in range((BATCH + 16) // 16):
            sel_local[pl.ds(k * 16, 16)] = zero

        @pl.when(jnp.logical_or(fg_base < NUM_FG, bg_base < NUM_BG))
        def _():
            def scan_body(i, carry):
                cps, cns = carry
                v = vchunk[pl.ds(i * 16, 16)]
                mask = v >= IOU_T
                mi = jnp.where(mask, 1, 0)
                csp = jnp.cumsum(mi)
                csn = jnp.cumsum(1 - mi)
                posd = cps + csp - 1
                negd = cns + csn - 1 + NUM_FG
                sel_pos = jnp.logical_and(mask, posd < NUM_FG)
                sel_neg = jnp.logical_and(~mask, negd < BATCH)
                selm = jnp.logical_or(sel_pos, sel_neg)
                dest = jnp.where(
                    sel_pos, posd, jnp.where(sel_neg, negd, BATCH + lane)
                )
                plsc.store_scatter(
                    sel_local, [dest], wid * ch + i * 16 + lane, mask=selm
                )
                npos = csp[15]
                return (cps + npos, cns + 16 - npos)

            lax.fori_loop(0, nv, scan_body, (fg_base, bg_base))

        pltpu.sync_copy(sel_local, sel_rows.at[pl.ds(wid * (BATCH + 16), BATCH + 16)])
        plsc.subcore_barrier()

        acc0 = zero
        acc1 = zero
        for r in range(NSUB):
            pltpu.sync_copy(
                sel_rows.at[pl.ds(r * (BATCH + 16) + wid * per_w, per_w)], rbuf
            )
            acc0 = acc0 + rbuf[pl.ds(0, 16)]
            acc1 = acc1 + rbuf[pl.ds(16, 16)]
        for b in range(per_w // 16):
            off = wid * per_w + b * 16
            jv = acc0 if b == 0 else acc1
            jv = jnp.clip(jv, 0, npad - 1)
            oidx = gath_i(perm_hbm, jv)
            oidx = jnp.clip(oidx, 0, npad - 1)
            ibuf[...] = oidx
            pltpu.sync_copy(ibuf, sidx.at[pl.ds(off, 16)])
            val = gath_f(vals_hbm, oidx)
            gidx = gath_i(idxs_hbm, oidx)
            gidx = jnp.clip(gidx, 0, GT_PAD - 1)
            gcls = gath_i(gtcls_hbm, gidx)
            cls = jnp.where(val >= IOU_T, gcls, NUM_CLASSES)
            ibuf[...] = cls
            pltpu.sync_copy(ibuf, scls.at[pl.ds(off, 16)])
            for r in range(4):
                bx = gath_f(bflat_hbm, oidx + r * npad)
                fbuf[...] = bx
                pltpu.sync_copy(fbuf, out_t.at[pl.ds(r * BATCH + off, 16)])
            for r in range(4):
                gx = gath_f(gtflat_hbm, gidx + r * GT_PAD)
                fbuf[...] = gx
                pltpu.sync_copy(fbuf, out_t.at[pl.ds((4 + r) * BATCH + off, 16)])
            fbuf[...] = val
            pltpu.sync_copy(fbuf, out_t.at[pl.ds(8 * BATCH + off, 16)])

    return sampler


def kernel(proposal_boxes, gt_boxes, gt_classes):
    n = proposal_boxes.shape[0]
    m = gt_boxes.shape[0]
    total = n + m
    blk = NSUB * C_BLK
    npad = ((total + blk - 1) // blk) * blk

    perm = jnp.asarray(_perm_const(total, npad, n))

    boxes_all = jnp.concatenate([proposal_boxes, gt_boxes], axis=0)
    bt = jnp.pad(boxes_all.T, ((0, 4), (0, npad - total)))
    btp = bt

    gt_pad = jnp.zeros((GT_PAD, 8), jnp.float32)
    gt_pad = gt_pad.at[:m, 0:4].set(gt_boxes)
    area_g = (gt_boxes[:, 2] - gt_boxes[:, 0]) * (gt_boxes[:, 3] - gt_boxes[:, 1])
    gtc = gt_pad.at[:m, 4].set(area_g)

    vals_p, idxs_p = _match(btp, gtc, npad)
    vals_p = vals_p.reshape(npad)
    idxs_p = idxs_p.reshape(npad)

    bflat = btp[:4].reshape(4 * npad)
    gtflat = gt_pad[:, 0:4].T.reshape(4 * GT_PAD)
    gtcls = jnp.zeros((GT_PAD,), jnp.int32).at[:m].set(gt_classes)

    sampler = _make_sampler(npad, m)
    out_t, sampled_idxs, sampled_cls = sampler(
        vals_p, idxs_p, perm, bflat, gtflat, gtcls
    )
    return out_t.reshape(9, BATCH).T, sampled_idxs, sampled_cls


# speedup vs baseline: 6.3981x; 1.2516x over previous
import functools

import jax
import jax.numpy as jnp
import numpy as np
from jax import lax
from jax.experimental import pallas as pl
from jax.experimental.pallas import tpu as pltpu
from jax.experimental.pallas import tpu_sc as plsc

NUM_CLASSES = 80
BATCH = 512
NUM_FG = 128
NUM_BG = 384
IOU_T = 0.5
C_BLK = 1024
GT_PAD = 512
NSUB = 16


def _threefry_uniform(seed, n):
    k1 = np.uint32(seed >> 32)
    k2 = np.uint32(seed & 0xFFFFFFFF)
    ks = [k1, k2, np.uint32(k1 ^ k2 ^ np.uint32(0x1BD11BDA))]
    x0 = np.zeros(n, np.uint32)
    x1 = np.arange(n, dtype=np.uint32)
    rotations = [[13, 15, 26, 6], [17, 29, 16, 24]]

    def rotl(v, d):
        return (v << np.uint32(d)) | (v >> np.uint32(32 - d))

    with np.errstate(over="ignore"):
        x0 = x0 + ks[0]
        x1 = x1 + ks[1]
        for i in range(5):
            for r in rotations[i % 2]:
                x0 = x0 + x1
                x1 = rotl(x1, r)
                x1 = x1 ^ x0
            x0 = x0 + ks[(i + 1) % 3]
            x1 = x1 + ks[(i + 2) % 3] + np.uint32(i + 1)
    bits = x0 ^ x1
    fl = ((bits >> np.uint32(9)) | np.uint32(0x3F800000)).view(np.float32)
    return np.maximum(np.float32(0.0), fl - np.float32(1.0))


@functools.lru_cache(maxsize=None)
def _perm_const(total, npad, gt0_pos):
    rnd = _threefry_uniform(42, total)
    perm = np.argsort(-rnd, kind="stable").astype(np.int32)
    pad = np.full((npad - total,), gt0_pos, np.int32)
    return np.concatenate([perm, pad])


def _match_body(bt_ref, gt_ref, vals_ref, idxs_ref):
    px1 = bt_ref[0:1, :]
    py1 = bt_ref[1:2, :]
    px2 = bt_ref[2:3, :]
    py2 = bt_ref[3:4, :]
    gx1 = gt_ref[:, 0:1]
    gy1 = gt_ref[:, 1:2]
    gx2 = gt_ref[:, 2:3]
    gy2 = gt_ref[:, 3:4]
    area_g = gt_ref[:, 4:5]
    area_p = (px2 - px1) * (py2 - py1)
    w = jnp.clip(jnp.minimum(gx2, px2) - jnp.maximum(gx1, px1), 0.0)
    h = jnp.clip(jnp.minimum(gy2, py2) - jnp.maximum(gy1, py1), 0.0)
    inter = w * h
    iou = inter / (area_g + area_p - inter)
    mx = jnp.max(iou, axis=0, keepdims=True)
    row = jax.lax.broadcasted_iota(jnp.int32, iou.shape, 0)
    idx = jnp.min(
        jnp.where(iou == mx, row, jnp.int32(GT_PAD)), axis=0, keepdims=True
    )
    vals_ref[...] = mx
    idxs_ref[...] = idx


def _match(btp, gtc, npad):
    grid = npad // C_BLK
    return pl.pallas_call(
        _match_body,
        grid=(grid,),
        in_specs=[
            pl.BlockSpec((8, C_BLK), lambda i: (0, i)),
            pl.BlockSpec((GT_PAD, 8), lambda i: (0, 0)),
        ],
        out_specs=[
            pl.BlockSpec((1, C_BLK), lambda i: (0, i)),
            pl.BlockSpec((1, C_BLK), lambda i: (0, i)),
        ],
        out_shape=[
            jax.ShapeDtypeStruct((1, npad), jnp.float32),
            jax.ShapeDtypeStruct((1, npad), jnp.int32),
        ],
    )(btp, gtc)


def _make_sampler(npad, m):
    ch = npad // NSUB
    nv = ch // 16
    ng = ch // 128
    per_w = BATCH // NSUB
    mesh = plsc.VectorSubcoreMesh(core_axis_name="c", subcore_axis_name="s")

    @functools.partial(
        pl.kernel,
        mesh=mesh,
        out_type=[
            jax.ShapeDtypeStruct((9 * BATCH,), jnp.float32),
            jax.ShapeDtypeStruct((BATCH,), jnp.int32),
            jax.ShapeDtypeStruct((BATCH,), jnp.int32),
        ],
        scratch_types=[
            pltpu.VMEM((ch,), jnp.int32),
            pltpu.VMEM((ch,), jnp.float32),
            pltpu.VMEM((16,), jnp.int32),
            pltpu.VMEM((16,), jnp.float32),
            pltpu.VMEM((NSUB, 16), jnp.int32),
            pltpu.VMEM((BATCH + 16,), jnp.int32),
            pltpu.VMEM((2 * 16,), jnp.int32),
            pltpu.VMEM_SHARED((NSUB, 16), jnp.int32),
            pltpu.VMEM_SHARED((NSUB * (BATCH + 16),), jnp.int32),
            pltpu.SemaphoreType.DMA,
        ],
        compiler_params=pltpu.CompilerParams(needs_layout_passes=False),
    )
    def sampler(
        vals_hbm,
        idxs_hbm,
        perm_hbm,
        bflat_hbm,
        gtflat_hbm,
        gtcls_hbm,
        out_t,
        sidx,
        scls,
        pchunk,
        vchunk,
        ibuf,
        fbuf,
        cbuf,
        sel_local,
        rbuf,
        shared,
        sel_rows,
        sem,
    ):
        wid = lax.axis_index("s")
        lane = lax.iota(jnp.int32, 16)

        def gath_f(tbl, idxv):
            pltpu.async_copy(tbl.at[idxv], fbuf, sem).wait()
            return fbuf[...]

        def gath_i(tbl, idxv):
            pltpu.async_copy(tbl.at[idxv], ibuf, sem).wait()
            return ibuf[...]

        pltpu.sync_copy(perm_hbm.at[pl.ds(wid * ch, ch)], pchunk)

        def gather_body(t, _):
            pltpu.async_copy(
                vals_hbm.at[pchunk.at[pl.ds(t * 128, 128)]],
                vchunk.at[pl.ds(t * 128, 128)],
                sem,
            ).wait()
            return 0

        lax.fori_loop(0, ng, gather_body, 0)

        def count_body(i, cp):
            v = vchunk[pl.ds(i * 16, 16)]
            return cp + jnp.sum(jnp.where(v >= IOU_T, 1, 0))

        cp = lax.fori_loop(0, nv, count_body, jnp.int32(0))
        cnt = jnp.where(lane == 0, cp, jnp.where(lane == 1, ch - cp, 0))
        ibuf[...] = cnt
        pltpu.sync_copy(ibuf, shared.at[wid])
        plsc.subcore_barrier()

        pltpu.sync_copy(shared, cbuf)
        acc = jnp.zeros((16,), jnp.int32)
        for r in range(NSUB):
            acc = acc + jnp.where(r < wid, cbuf[r, :], 0)
        fg_base = acc[0]
        bg_base = acc[1]

        zero = jnp.zeros((16,), jnp.int32)
        for k in range((BATCH + 16) // 16):
            sel_local[pl.ds(k * 16, 16)] = zero

        @pl.when(jnp.logical_or(fg_base < NUM_FG, bg_base < NUM_BG))
        def _():
            def scan_body(i, carry):
                cps, cns = carry
                v = vchunk[pl.ds(i * 16, 16)]
                mask = v >= IOU_T
                mi = jnp.where(mask, 1, 0)
                csp = jnp.cumsum(mi)
                csn = jnp.cumsum(1 - mi)
                posd = cps + csp - 1
                negd = cns + csn - 1 + NUM_FG
                sel_pos = jnp.logical_and(mask, posd < NUM_FG)
                sel_neg = jnp.logical_and(~mask, negd < BATCH)
                selm = jnp.logical_or(sel_pos, sel_neg)
                dest = jnp.where(
                    sel_pos, posd, jnp.where(sel_neg, negd, BATCH + lane)
                )
                plsc.store_scatter(
                    sel_local, [dest], wid * ch + i * 16 + lane, mask=selm
                )
                npos = csp[15]
                return (cps + npos, cns + 16 - npos)

            lax.fori_loop(0, nv, scan_body, (fg_base, bg_base))

        pltpu.sync_copy(sel_local, sel_rows.at[pl.ds(wid * (BATCH + 16), BATCH + 16)])
        plsc.subcore_barrier()

        acc0 = zero
        acc1 = zero
        for r in range(NSUB):
            pltpu.sync_copy(
                sel_rows.at[pl.ds(r * (BATCH + 16) + wid * per_w, per_w)], rbuf
            )
            acc0 = acc0 + rbuf[pl.ds(0, 16)]
            acc1 = acc1 + rbuf[pl.ds(16, 16)]
        for b in range(per_w // 16):
            off = wid * per_w + b * 16
            jv = acc0 if b == 0 else acc1
            jv = jnp.clip(jv, 0, npad - 1)
            oidx = gath_i(perm_hbm, jv)
            oidx = jnp.clip(oidx, 0, npad - 1)
            ibuf[...] = oidx
            pltpu.sync_copy(ibuf, sidx.at[pl.ds(off, 16)])
            val = gath_f(vals_hbm, oidx)
            gidx = gath_i(idxs_hbm, oidx)
            gidx = jnp.clip(gidx, 0, GT_PAD - 1)
            gcls = gath_i(gtcls_hbm, gidx)
            cls = jnp.where(val >= IOU_T, gcls, NUM_CLASSES)
            ibuf[...] = cls
            pltpu.sync_copy(ibuf, scls.at[pl.ds(off, 16)])
            for r in range(4):
                bx = gath_f(bflat_hbm, oidx + r * npad)
                fbuf[...] = bx
                pltpu.sync_copy(fbuf, out_t.at[pl.ds(r * BATCH + off, 16)])
            for r in range(4):
                gx = gath_f(gtflat_hbm, gidx + r * GT_PAD)
                fbuf[...] = gx
                pltpu.sync_copy(fbuf, out_t.at[pl.ds((4 + r) * BATCH + off, 16)])
            fbuf[...] = val
            pltpu.sync_copy(fbuf, out_t.at[pl.ds(8 * BATCH + off, 16)])

    return sampler


def kernel(proposal_boxes, gt_boxes, gt_classes):
    n = proposal_boxes.shape[0]
    m = gt_boxes.shape[0]
    total = n + m
    blk = NSUB * 128
    assert blk % C_BLK == 0 or C_BLK % blk == 0
    blk = max(blk, C_BLK)
    npad = ((total + blk - 1) // blk) * blk

    perm = jnp.asarray(_perm_const(total, npad, n))

    boxes_all = jnp.concatenate([proposal_boxes, gt_boxes], axis=0)
    bt = jnp.pad(boxes_all.T, ((0, 4), (0, npad - total)))
    btp = bt

    gt_pad = jnp.zeros((GT_PAD, 8), jnp.float32)
    gt_pad = gt_pad.at[:m, 0:4].set(gt_boxes)
    area_g = (gt_boxes[:, 2] - gt_boxes[:, 0]) * (gt_boxes[:, 3] - gt_boxes[:, 1])
    gtc = gt_pad.at[:m, 4].set(area_g)

    vals_p, idxs_p = _match(btp, gtc, npad)
    vals_p = vals_p.reshape(npad)
    idxs_p = idxs_p.reshape(npad)

    bflat = btp[:4].reshape(4 * npad)
    gtflat = gt_pad[:, 0:4].T.reshape(4 * GT_PAD)
    gtcls = jnp.zeros((GT_PAD,), jnp.int32).at[:m].set(gt_classes)

    sampler = _make_sampler(npad, m)
    out_t, sampled_idxs, sampled_cls = sampler(
        vals_p, idxs_p, perm, bflat, gtflat, gtcls
    )
    return out_t.reshape(9, BATCH).T, sampled_idxs, sampled_cls
